# Initial kernel scaffold; baseline (speedup 1.0000x reference)
import jax
import jax.numpy as jnp
from jax.experimental import pallas as pl


def kernel(x, expert_ids, W, b, lora_a, lora_b):
    raise NotImplementedError("write your pallas kernel here")


# speedup vs baseline: 8.9265x; 8.9265x over previous
import functools

import jax
import jax.numpy as jnp
from jax.experimental import pallas as pl

NUM_TOKENS = 8192
D_IN = 2048
D_OUT = 2048
RANK = 16
NUM_EXPERTS = 16
SCALING = 32 / float(RANK)

TOKEN_BLOCK = 512


def _fused_kernel(x_ref, eid_ref, w_ref, b_ref, a_ref, bt_ref, o_ref):
    x = x_ref[...]
    base = jax.lax.dot_general(
        x, w_ref[...], (((1,), (1,)), ((), ())),
        preferred_element_type=jnp.float32)
    inter = jax.lax.dot_general(
        x, a_ref[...], (((1,), (1,)), ((), ())),
        preferred_element_type=jnp.float32)
    eids = eid_ref[...]
    col_expert = jax.lax.broadcasted_iota(
        jnp.int32, inter.shape, 1) // RANK
    inter = jnp.where(col_expert == eids, inter, 0.0)
    delta = jax.lax.dot_general(
        inter, bt_ref[...], (((1,), (0,)), ((), ())),
        preferred_element_type=jnp.float32)
    o_ref[...] = base + b_ref[...] + SCALING * delta


@functools.partial(jax.jit, static_argnames=())
def kernel(x, expert_ids, W, b, lora_a, lora_b):
    n_tokens = x.shape[0]
    eids = expert_ids.astype(jnp.int32).reshape(n_tokens, 1)
    a_flat = lora_a.reshape(NUM_EXPERTS * RANK, D_IN)
    bt = lora_b.transpose(0, 2, 1).reshape(NUM_EXPERTS * RANK, D_OUT)
    b2 = b.reshape(1, D_OUT)

    grid = (n_tokens // TOKEN_BLOCK,)
    out = pl.pallas_call(
        _fused_kernel,
        grid=grid,
        in_specs=[
            pl.BlockSpec((TOKEN_BLOCK, D_IN), lambda i: (i, 0)),
            pl.BlockSpec((TOKEN_BLOCK, 1), lambda i: (i, 0)),
            pl.BlockSpec((D_OUT, D_IN), lambda i: (0, 0)),
            pl.BlockSpec((1, D_OUT), lambda i: (0, 0)),
            pl.BlockSpec((NUM_EXPERTS * RANK, D_IN), lambda i: (0, 0)),
            pl.BlockSpec((NUM_EXPERTS * RANK, D_OUT), lambda i: (0, 0)),
        ],
        out_specs=pl.BlockSpec((TOKEN_BLOCK, D_OUT), lambda i: (i, 0)),
        out_shape=jax.ShapeDtypeStruct((n_tokens, D_OUT), jnp.float32),
    )(x, eids, W, b2, a_flat, bt)
    return out
